# Initial kernel scaffold; baseline (speedup 1.0000x reference)
import jax
import jax.numpy as jnp
from jax.experimental import pallas as pl
from jax.experimental.pallas import tpu as pltpu

_NE = 1024
_ED = 64
_BETA = 0.25
_B = 8
_HW = 64
_HCH = 8
_TOK = 512
_NBLK = _B * (_HW // _HCH)
_NTOK = _B * _HW * _HW
_NELEM = _NTOK * _ED


def _vq_body(z_ref, w_ref, idx_ref, zq_ref, loss_ref, perp_ref,
             hist_ref, sq_ref):
    step = pl.program_id(0) * (_HW // _HCH) + pl.program_id(1)

    zb = z_ref[0]
    zt = jnp.transpose(zb.reshape(_ED, _TOK))

    w = w_ref[...]
    wsq = jnp.sum(w * w, axis=1)
    s = jax.lax.dot_general(zt, w, (((1,), (1,)), ((), ())),
                            preferred_element_type=jnp.float32)
    d = wsq[None, :] - 2.0 * s
    idx = jnp.argmin(d, axis=1).astype(jnp.int32)
    idx_ref[0, 0, :] = idx

    iota = jax.lax.broadcasted_iota(jnp.int32, (_TOK, _NE), 1)
    oh = (idx[:, None] == iota).astype(jnp.float32)
    zq = jnp.dot(oh, w, preferred_element_type=jnp.float32)
    zq_ref[0] = jnp.transpose(zq).reshape(_ED, _HCH, _HW)

    colsum = jnp.sum(oh, axis=0)
    sqblk = jnp.sum((zq - zt) ** 2)

    @pl.when(step == 0)
    def _init():
        hist_ref[0, :] = colsum
        sq_ref[0] = sqblk

    @pl.when(step > 0)
    def _acc():
        hist_ref[0, :] = hist_ref[0, :] + colsum
        sq_ref[0] = sq_ref[0] + sqblk

    @pl.when(step == _NBLK - 1)
    def _fin():
        loss_ref[0, 0] = (1.0 + _BETA) * sq_ref[0] / float(_NELEM)
        e = hist_ref[0, :] / float(_NTOK)
        perp_ref[0, 0] = jnp.exp(-jnp.sum(e * jnp.log(e + 1e-10)))


def _quantize_one(z, w):
    grid = (_B, _HW // _HCH)
    out = pl.pallas_call(
        _vq_body,
        grid=grid,
        in_specs=[
            pl.BlockSpec((1, _ED, _HCH, _HW), lambda b, h: (b, 0, h, 0)),
            pl.BlockSpec((_NE, _ED), lambda b, h: (0, 0)),
        ],
        out_specs=[
            pl.BlockSpec((1, 1, _TOK), lambda b, h: (b * (_HW // _HCH) + h, 0, 0)),
            pl.BlockSpec((1, _ED, _HCH, _HW), lambda b, h: (b, 0, h, 0)),
            pl.BlockSpec((1, 1), lambda b, h: (0, 0)),
            pl.BlockSpec((1, 1), lambda b, h: (0, 0)),
        ],
        out_shape=[
            jax.ShapeDtypeStruct((_NBLK, 1, _TOK), jnp.int32),
            jax.ShapeDtypeStruct((_B, _ED, _HW, _HW), jnp.float32),
            jax.ShapeDtypeStruct((1, 1), jnp.float32),
            jax.ShapeDtypeStruct((1, 1), jnp.float32),
        ],
        scratch_shapes=[
            pltpu.VMEM((1, _NE), jnp.float32),
            pltpu.SMEM((1,), jnp.float32),
        ],
    )(z, w)
    idx, zq, loss, perp = out
    return idx.reshape(_NTOK), zq, loss[0, 0], perp[0, 0]


def kernel(z0, z1, W_z, W):
    i0, zq0, l0, p0 = _quantize_one(z0, W_z)
    i1, zq1, l1, p1 = _quantize_one(z1, W)
    return (l0 + l1, zq0, zq1, (p0 + p1) / 2.0, i0, i1)


# speedup vs baseline: 3.0068x; 3.0068x over previous
import jax
import jax.numpy as jnp
from jax.experimental import pallas as pl
from jax.experimental.pallas import tpu as pltpu

_NE = 1024
_ED = 64
_BETA = 0.25
_B = 8
_HW = 64
_TOK = 512
_NTOK = _B * _HW * _HW
_NBLK = _NTOK // _TOK
_NELEM = _NTOK * _ED


def _vq_body(zt_ref, w_ref, zsq_ref, wsq_ref, idx_ref, zq_ref, loss_ref,
             perp_ref, hist_ref, sq_ref):
    step = pl.program_id(0)

    zt = zt_ref[...]
    w = w_ref[...]
    wsq = wsq_ref[0]
    zsq = zsq_ref[...]
    s = jax.lax.dot_general(zt, w, (((1,), (1,)), ((), ())),
                            preferred_element_type=jnp.float32)
    d = (zsq + wsq[None, :]) - 2.0 * s
    iota = jax.lax.broadcasted_iota(jnp.int32, (_TOK, _NE), 1)
    dmin = jnp.min(d, axis=1, keepdims=True)
    idx = jnp.min(jnp.where(d == dmin, iota, _NE), axis=1).astype(jnp.int32)
    idx_ref[0, 0, :] = idx

    oh = (idx[:, None] == iota).astype(jnp.float32)
    zq = jnp.dot(oh, w, preferred_element_type=jnp.float32)
    zq_ref[0] = jnp.transpose(zq).reshape(_ED, _TOK // _HW, _HW)

    colsum = jnp.sum(oh, axis=0)
    sqblk = jnp.sum((zq - zt) ** 2)

    @pl.when(step == 0)
    def _init():
        hist_ref[0, :] = colsum
        sq_ref[0] = sqblk

    @pl.when(step > 0)
    def _acc():
        hist_ref[0, :] = hist_ref[0, :] + colsum
        sq_ref[0] = sq_ref[0] + sqblk

    @pl.when(step == _NBLK - 1)
    def _fin():
        loss = (1.0 + _BETA) * sq_ref[0] / float(_NELEM)
        loss_ref[...] = jnp.full((1, 1), loss, dtype=jnp.float32)
        e = hist_ref[0, :] / float(_NTOK)
        perp = jnp.exp(-jnp.sum(e * jnp.log(e + 1e-10)))
        perp_ref[...] = jnp.full((1, 1), perp, dtype=jnp.float32)


_HPB = _TOK // _HW


def _quantize_one(z, w):
    zt = jnp.transpose(z, (0, 2, 3, 1)).reshape(_NTOK, _ED)
    zsq = jnp.sum(zt ** 2, axis=1, keepdims=True)
    wsq = jnp.sum(w ** 2, axis=1).reshape(1, _NE)
    out = pl.pallas_call(
        _vq_body,
        grid=(_NBLK,),
        in_specs=[
            pl.BlockSpec((_TOK, _ED), lambda i: (i, 0)),
            pl.BlockSpec((_NE, _ED), lambda i: (0, 0)),
            pl.BlockSpec((_TOK, 1), lambda i: (i, 0)),
            pl.BlockSpec((1, _NE), lambda i: (0, 0)),
        ],
        out_specs=[
            pl.BlockSpec((1, 1, _TOK), lambda i: (i, 0, 0)),
            pl.BlockSpec((1, _ED, _HPB, _HW),
                         lambda i: (i // (_HW // _HPB), 0, i % (_HW // _HPB), 0)),
            pl.BlockSpec((1, 1), lambda i: (0, 0)),
            pl.BlockSpec((1, 1), lambda i: (0, 0)),
        ],
        out_shape=[
            jax.ShapeDtypeStruct((_NBLK, 1, _TOK), jnp.int32),
            jax.ShapeDtypeStruct((_B, _ED, _HW, _HW), jnp.float32),
            jax.ShapeDtypeStruct((1, 1), jnp.float32),
            jax.ShapeDtypeStruct((1, 1), jnp.float32),
        ],
        scratch_shapes=[
            pltpu.VMEM((1, _NE), jnp.float32),
            pltpu.SMEM((1,), jnp.float32),
        ],
    )(zt, w, zsq, wsq)
    idx, zq, loss, perp = out
    return idx.reshape(_NTOK), zq, loss[0, 0], perp[0, 0]


def kernel(z0, z1, W_z, W):
    i0, zq0, l0, p0 = _quantize_one(z0, W_z)
    i1, zq1, l1, p1 = _quantize_one(z1, W)
    return (l0 + l1, zq0, zq1, (p0 + p1) / 2.0, i0, i1)


# speedup vs baseline: 3.2693x; 1.0873x over previous
import jax
import jax.numpy as jnp
from jax.experimental import pallas as pl
from jax.experimental.pallas import tpu as pltpu

_NE = 1024
_ED = 64
_BETA = 0.25
_B = 8
_HW = 64
_TOK = 1024
_NTOK = _B * _HW * _HW
_NBLK = _NTOK // _TOK
_NELEM = _NTOK * _ED


def _vq_body(zt_ref, w_ref, zsq_ref, wsq_ref, idx_ref, zq_ref, loss_ref,
             perp_ref, hist_ref, sq_ref):
    step = pl.program_id(0)

    zt = zt_ref[...]
    w = w_ref[...]
    wsq = wsq_ref[0]
    zsq = zsq_ref[...]
    s = jax.lax.dot_general(zt, w, (((1,), (1,)), ((), ())),
                            preferred_element_type=jnp.float32)
    d = (zsq + wsq[None, :]) - 2.0 * s
    iota = jax.lax.broadcasted_iota(jnp.int32, (_TOK, _NE), 1)
    dmin = jnp.min(d, axis=1, keepdims=True)
    idx = jnp.min(jnp.where(d == dmin, iota, _NE), axis=1).astype(jnp.int32)
    idx_ref[0, 0, :] = idx

    oh = (idx[:, None] == iota).astype(jnp.float32)
    zq = jnp.dot(oh, w, preferred_element_type=jnp.float32)
    zq_ref[0] = jnp.transpose(zq).reshape(_ED, _TOK // _HW, _HW)

    colsum = jnp.sum(oh, axis=0)
    sqblk = jnp.sum((zq - zt) ** 2)

    @pl.when(step == 0)
    def _init():
        hist_ref[0, :] = colsum
        sq_ref[0] = sqblk

    @pl.when(step > 0)
    def _acc():
        hist_ref[0, :] = hist_ref[0, :] + colsum
        sq_ref[0] = sq_ref[0] + sqblk

    @pl.when(step == _NBLK - 1)
    def _fin():
        loss = (1.0 + _BETA) * sq_ref[0] / float(_NELEM)
        loss_ref[...] = jnp.full((1, 1), loss, dtype=jnp.float32)
        e = hist_ref[0, :] / float(_NTOK)
        perp = jnp.exp(-jnp.sum(e * jnp.log(e + 1e-10)))
        perp_ref[...] = jnp.full((1, 1), perp, dtype=jnp.float32)


_HPB = _TOK // _HW


def _quantize_one(z, w):
    zt = jnp.transpose(z, (0, 2, 3, 1)).reshape(_NTOK, _ED)
    zsq = jnp.sum(zt ** 2, axis=1, keepdims=True)
    wsq = jnp.sum(w ** 2, axis=1).reshape(1, _NE)
    out = pl.pallas_call(
        _vq_body,
        grid=(_NBLK,),
        in_specs=[
            pl.BlockSpec((_TOK, _ED), lambda i: (i, 0)),
            pl.BlockSpec((_NE, _ED), lambda i: (0, 0)),
            pl.BlockSpec((_TOK, 1), lambda i: (i, 0)),
            pl.BlockSpec((1, _NE), lambda i: (0, 0)),
        ],
        out_specs=[
            pl.BlockSpec((1, 1, _TOK), lambda i: (i, 0, 0)),
            pl.BlockSpec((1, _ED, _HPB, _HW),
                         lambda i: (i // (_HW // _HPB), 0, i % (_HW // _HPB), 0)),
            pl.BlockSpec((1, 1), lambda i: (0, 0)),
            pl.BlockSpec((1, 1), lambda i: (0, 0)),
        ],
        out_shape=[
            jax.ShapeDtypeStruct((_NBLK, 1, _TOK), jnp.int32),
            jax.ShapeDtypeStruct((_B, _ED, _HW, _HW), jnp.float32),
            jax.ShapeDtypeStruct((1, 1), jnp.float32),
            jax.ShapeDtypeStruct((1, 1), jnp.float32),
        ],
        scratch_shapes=[
            pltpu.VMEM((1, _NE), jnp.float32),
            pltpu.SMEM((1,), jnp.float32),
        ],
    )(zt, w, zsq, wsq)
    idx, zq, loss, perp = out
    return idx.reshape(_NTOK), zq, loss[0, 0], perp[0, 0]


def kernel(z0, z1, W_z, W):
    i0, zq0, l0, p0 = _quantize_one(z0, W_z)
    i1, zq1, l1, p1 = _quantize_one(z1, W)
    return (l0 + l1, zq0, zq1, (p0 + p1) / 2.0, i0, i1)


# speedup vs baseline: 3.4814x; 1.0649x over previous
import jax
import jax.numpy as jnp
from jax.experimental import pallas as pl
from jax.experimental.pallas import tpu as pltpu

_NE = 1024
_ED = 64
_BETA = 0.25
_B = 8
_HW = 64
_TOK = 2048
_NTOK = _B * _HW * _HW
_NBLK = _NTOK // _TOK
_NELEM = _NTOK * _ED


def _vq_body(zt_ref, w_ref, zsq_ref, wsq_ref, idx_ref, zq_ref, loss_ref,
             perp_ref, hist_ref, sq_ref):
    step = pl.program_id(0)

    zt = zt_ref[...]
    w = w_ref[...]
    wsq = wsq_ref[0]
    zsq = zsq_ref[...]
    s = jax.lax.dot_general(zt, w, (((1,), (1,)), ((), ())),
                            preferred_element_type=jnp.float32)
    d = (zsq + wsq[None, :]) - 2.0 * s
    iota = jax.lax.broadcasted_iota(jnp.int32, (_TOK, _NE), 1)
    dmin = jnp.min(d, axis=1, keepdims=True)
    idx = jnp.min(jnp.where(d == dmin, iota, _NE), axis=1).astype(jnp.int32)
    idx_ref[0, 0, :] = idx

    oh = (idx[:, None] == iota).astype(jnp.float32)
    zq = jnp.dot(oh, w, preferred_element_type=jnp.float32)
    zq_ref[0] = jnp.transpose(zq).reshape(_ED, _TOK // _HW, _HW)

    colsum = jnp.sum(oh, axis=0)
    sqblk = jnp.sum((zq - zt) ** 2)

    @pl.when(step == 0)
    def _init():
        hist_ref[0, :] = colsum
        sq_ref[0] = sqblk

    @pl.when(step > 0)
    def _acc():
        hist_ref[0, :] = hist_ref[0, :] + colsum
        sq_ref[0] = sq_ref[0] + sqblk

    @pl.when(step == _NBLK - 1)
    def _fin():
        loss = (1.0 + _BETA) * sq_ref[0] / float(_NELEM)
        loss_ref[...] = jnp.full((1, 1), loss, dtype=jnp.float32)
        e = hist_ref[0, :] / float(_NTOK)
        perp = jnp.exp(-jnp.sum(e * jnp.log(e + 1e-10)))
        perp_ref[...] = jnp.full((1, 1), perp, dtype=jnp.float32)


_HPB = _TOK // _HW


def _quantize_one(z, w):
    zt = jnp.transpose(z, (0, 2, 3, 1)).reshape(_NTOK, _ED)
    zsq = jnp.sum(zt ** 2, axis=1, keepdims=True)
    wsq = jnp.sum(w ** 2, axis=1).reshape(1, _NE)
    out = pl.pallas_call(
        _vq_body,
        grid=(_NBLK,),
        in_specs=[
            pl.BlockSpec((_TOK, _ED), lambda i: (i, 0)),
            pl.BlockSpec((_NE, _ED), lambda i: (0, 0)),
            pl.BlockSpec((_TOK, 1), lambda i: (i, 0)),
            pl.BlockSpec((1, _NE), lambda i: (0, 0)),
        ],
        out_specs=[
            pl.BlockSpec((1, 1, _TOK), lambda i: (i, 0, 0)),
            pl.BlockSpec((1, _ED, _HPB, _HW),
                         lambda i: (i // (_HW // _HPB), 0, i % (_HW // _HPB), 0)),
            pl.BlockSpec((1, 1), lambda i: (0, 0)),
            pl.BlockSpec((1, 1), lambda i: (0, 0)),
        ],
        out_shape=[
            jax.ShapeDtypeStruct((_NBLK, 1, _TOK), jnp.int32),
            jax.ShapeDtypeStruct((_B, _ED, _HW, _HW), jnp.float32),
            jax.ShapeDtypeStruct((1, 1), jnp.float32),
            jax.ShapeDtypeStruct((1, 1), jnp.float32),
        ],
        scratch_shapes=[
            pltpu.VMEM((1, _NE), jnp.float32),
            pltpu.SMEM((1,), jnp.float32),
        ],
    )(zt, w, zsq, wsq)
    idx, zq, loss, perp = out
    return idx.reshape(_NTOK), zq, loss[0, 0], perp[0, 0]


def kernel(z0, z1, W_z, W):
    i0, zq0, l0, p0 = _quantize_one(z0, W_z)
    i1, zq1, l1, p1 = _quantize_one(z1, W)
    return (l0 + l1, zq0, zq1, (p0 + p1) / 2.0, i0, i1)


# speedup vs baseline: 3.5494x; 1.0196x over previous
import jax
import jax.numpy as jnp
from jax.experimental import pallas as pl
from jax.experimental.pallas import tpu as pltpu

_NE = 1024
_ED = 64
_BETA = 0.25
_B = 8
_HW = 64
_TOK = 4096
_NTOK = _B * _HW * _HW
_NBLK = _NTOK // _TOK
_NELEM = _NTOK * _ED


def _vq_body(zt_ref, w_ref, zsq_ref, wsq_ref, idx_ref, zq_ref, loss_ref,
             perp_ref, hist_ref, sq_ref):
    step = pl.program_id(0)

    zt = zt_ref[...]
    w = w_ref[...]
    wsq = wsq_ref[0]
    zsq = zsq_ref[...]
    s = jax.lax.dot_general(zt, w, (((1,), (1,)), ((), ())),
                            preferred_element_type=jnp.float32)
    d = (zsq + wsq[None, :]) - 2.0 * s
    iota = jax.lax.broadcasted_iota(jnp.int32, (_TOK, _NE), 1)
    dmin = jnp.min(d, axis=1, keepdims=True)
    idx = jnp.min(jnp.where(d == dmin, iota, _NE), axis=1).astype(jnp.int32)
    idx_ref[0, 0, :] = idx

    oh = (idx[:, None] == iota).astype(jnp.float32)
    zq = jnp.dot(oh, w, preferred_element_type=jnp.float32)
    zq_ref[0] = jnp.transpose(zq).reshape(_ED, _TOK // _HW, _HW)

    colsum = jnp.sum(oh, axis=0)
    sqblk = jnp.sum((zq - zt) ** 2)

    @pl.when(step == 0)
    def _init():
        hist_ref[0, :] = colsum
        sq_ref[0] = sqblk

    @pl.when(step > 0)
    def _acc():
        hist_ref[0, :] = hist_ref[0, :] + colsum
        sq_ref[0] = sq_ref[0] + sqblk

    @pl.when(step == _NBLK - 1)
    def _fin():
        loss = (1.0 + _BETA) * sq_ref[0] / float(_NELEM)
        loss_ref[...] = jnp.full((1, 1), loss, dtype=jnp.float32)
        e = hist_ref[0, :] / float(_NTOK)
        perp = jnp.exp(-jnp.sum(e * jnp.log(e + 1e-10)))
        perp_ref[...] = jnp.full((1, 1), perp, dtype=jnp.float32)


_HPB = _TOK // _HW


def _quantize_one(z, w):
    zt = jnp.transpose(z, (0, 2, 3, 1)).reshape(_NTOK, _ED)
    zsq = jnp.sum(zt ** 2, axis=1, keepdims=True)
    wsq = jnp.sum(w ** 2, axis=1).reshape(1, _NE)
    out = pl.pallas_call(
        _vq_body,
        grid=(_NBLK,),
        in_specs=[
            pl.BlockSpec((_TOK, _ED), lambda i: (i, 0)),
            pl.BlockSpec((_NE, _ED), lambda i: (0, 0)),
            pl.BlockSpec((_TOK, 1), lambda i: (i, 0)),
            pl.BlockSpec((1, _NE), lambda i: (0, 0)),
        ],
        out_specs=[
            pl.BlockSpec((1, 1, _TOK), lambda i: (i, 0, 0)),
            pl.BlockSpec((1, _ED, _HPB, _HW),
                         lambda i: (i // (_HW // _HPB), 0, i % (_HW // _HPB), 0)),
            pl.BlockSpec((1, 1), lambda i: (0, 0)),
            pl.BlockSpec((1, 1), lambda i: (0, 0)),
        ],
        out_shape=[
            jax.ShapeDtypeStruct((_NBLK, 1, _TOK), jnp.int32),
            jax.ShapeDtypeStruct((_B, _ED, _HW, _HW), jnp.float32),
            jax.ShapeDtypeStruct((1, 1), jnp.float32),
            jax.ShapeDtypeStruct((1, 1), jnp.float32),
        ],
        scratch_shapes=[
            pltpu.VMEM((1, _NE), jnp.float32),
            pltpu.SMEM((1,), jnp.float32),
        ],
    )(zt, w, zsq, wsq)
    idx, zq, loss, perp = out
    return idx.reshape(_NTOK), zq, loss[0, 0], perp[0, 0]


def kernel(z0, z1, W_z, W):
    i0, zq0, l0, p0 = _quantize_one(z0, W_z)
    i1, zq1, l1, p1 = _quantize_one(z1, W)
    return (l0 + l1, zq0, zq1, (p0 + p1) / 2.0, i0, i1)


# speedup vs baseline: 3.7799x; 1.0649x over previous
import jax
import jax.numpy as jnp
from jax.experimental import pallas as pl
from jax.experimental.pallas import tpu as pltpu

_NE = 1024
_ED = 64
_BETA = 0.25
_B = 8
_HW = 64
_TOK = 4096
_NTOK = _B * _HW * _HW
_NBLK = _NTOK // _TOK
_NELEM = _NTOK * _ED


def _vq_body(zt_ref, w_ref, zsq_ref, wsq_ref, idx_ref, zq_ref, loss_ref,
             perp_ref, hist_ref, sq_ref):
    step = pl.program_id(0)

    zt = zt_ref[...]
    w = w_ref[...]
    wsq = wsq_ref[0]
    zsq = zsq_ref[...]
    s = jax.lax.dot_general(zt, w, (((1,), (1,)), ((), ())),
                            preferred_element_type=jnp.float32)
    d = (zsq + wsq[None, :]) - 2.0 * s
    fiota = jax.lax.broadcasted_iota(jnp.int32, (_TOK, _NE), 1).astype(jnp.float32)
    dmin = jnp.min(d, axis=1, keepdims=True)
    masked = jnp.where(d == dmin, fiota, jnp.float32(_NE))
    fidx = jnp.min(masked, axis=1, keepdims=True)
    idx = fidx[:, 0].astype(jnp.int32)
    idx_ref[0, 0, :] = idx

    oh = (masked == fidx).astype(jnp.float32)
    zq = jnp.dot(oh, w, preferred_element_type=jnp.float32)
    zq_ref[0] = jnp.transpose(zq).reshape(_ED, _TOK // _HW, _HW)

    colsum = jnp.sum(oh, axis=0)
    sqblk = jnp.sum((zq - zt) ** 2)

    @pl.when(step == 0)
    def _init():
        hist_ref[0, :] = colsum
        sq_ref[0] = sqblk

    @pl.when(step > 0)
    def _acc():
        hist_ref[0, :] = hist_ref[0, :] + colsum
        sq_ref[0] = sq_ref[0] + sqblk

    @pl.when(step == _NBLK - 1)
    def _fin():
        loss = (1.0 + _BETA) * sq_ref[0] / float(_NELEM)
        loss_ref[...] = jnp.full((1, 1), loss, dtype=jnp.float32)
        e = hist_ref[0, :] / float(_NTOK)
        perp = jnp.exp(-jnp.sum(e * jnp.log(e + 1e-10)))
        perp_ref[...] = jnp.full((1, 1), perp, dtype=jnp.float32)


_HPB = _TOK // _HW


def _quantize_one(z, w):
    zt = jnp.transpose(z, (0, 2, 3, 1)).reshape(_NTOK, _ED)
    zsq = jnp.sum(zt ** 2, axis=1, keepdims=True)
    wsq = jnp.sum(w ** 2, axis=1).reshape(1, _NE)
    out = pl.pallas_call(
        _vq_body,
        grid=(_NBLK,),
        in_specs=[
            pl.BlockSpec((_TOK, _ED), lambda i: (i, 0)),
            pl.BlockSpec((_NE, _ED), lambda i: (0, 0)),
            pl.BlockSpec((_TOK, 1), lambda i: (i, 0)),
            pl.BlockSpec((1, _NE), lambda i: (0, 0)),
        ],
        out_specs=[
            pl.BlockSpec((1, 1, _TOK), lambda i: (i, 0, 0)),
            pl.BlockSpec((1, _ED, _HPB, _HW),
                         lambda i: (i // (_HW // _HPB), 0, i % (_HW // _HPB), 0)),
            pl.BlockSpec((1, 1), lambda i: (0, 0)),
            pl.BlockSpec((1, 1), lambda i: (0, 0)),
        ],
        out_shape=[
            jax.ShapeDtypeStruct((_NBLK, 1, _TOK), jnp.int32),
            jax.ShapeDtypeStruct((_B, _ED, _HW, _HW), jnp.float32),
            jax.ShapeDtypeStruct((1, 1), jnp.float32),
            jax.ShapeDtypeStruct((1, 1), jnp.float32),
        ],
        scratch_shapes=[
            pltpu.VMEM((1, _NE), jnp.float32),
            pltpu.SMEM((1,), jnp.float32),
        ],
    )(zt, w, zsq, wsq)
    idx, zq, loss, perp = out
    return idx.reshape(_NTOK), zq, loss[0, 0], perp[0, 0]


def kernel(z0, z1, W_z, W):
    i0, zq0, l0, p0 = _quantize_one(z0, W_z)
    i1, zq1, l1, p1 = _quantize_one(z1, W)
    return (l0 + l1, zq0, zq1, (p0 + p1) / 2.0, i0, i1)


# speedup vs baseline: 4.0704x; 1.0768x over previous
import jax
import jax.numpy as jnp
from jax.experimental import pallas as pl
from jax.experimental.pallas import tpu as pltpu

_NE = 1024
_ED = 64
_BETA = 0.25
_B = 8
_HW = 64
_TOK = 4096
_NTOK = _B * _HW * _HW
_NBLK = _NTOK // _TOK
_NELEM = _NTOK * _ED


def _vq_body(zt_ref, w_ref, zsq_ref, wsq_ref, idx_ref, zq_ref, loss_ref,
             perp_ref, hist_ref, sq_ref):
    step = pl.program_id(0)

    zt = zt_ref[...]
    w = w_ref[...]
    wsq = wsq_ref[0]
    zsq = zsq_ref[...]
    s2 = jax.lax.dot_general(zt, -2.0 * w, (((1,), (1,)), ((), ())),
                             preferred_element_type=jnp.float32)
    d = (zsq + wsq[None, :]) + s2
    fiota = jax.lax.broadcasted_iota(jnp.int32, (_TOK, _NE), 1).astype(jnp.float32)
    dmin = jnp.min(d, axis=1, keepdims=True)
    masked = jnp.where(d == dmin, fiota, jnp.float32(_NE))
    fidx = jnp.min(masked, axis=1, keepdims=True)
    idx = fidx[:, 0].astype(jnp.int32)
    idx_ref[0, 0, :] = idx

    oh = (masked == fidx).astype(jnp.float32)
    zq = jnp.dot(oh, w, preferred_element_type=jnp.float32)
    zq_ref[0] = jnp.transpose(zq).reshape(_ED, _TOK // _HW, _HW)

    colsum = jnp.dot(jnp.ones((1, _TOK), jnp.float32), oh,
                     preferred_element_type=jnp.float32)[0]
    sqblk = jnp.sum((zq - zt) ** 2)

    @pl.when(step == 0)
    def _init():
        hist_ref[0, :] = colsum
        sq_ref[0] = sqblk

    @pl.when(step > 0)
    def _acc():
        hist_ref[0, :] = hist_ref[0, :] + colsum
        sq_ref[0] = sq_ref[0] + sqblk

    @pl.when(step == _NBLK - 1)
    def _fin():
        loss = (1.0 + _BETA) * sq_ref[0] / float(_NELEM)
        loss_ref[...] = jnp.full((1, 1), loss, dtype=jnp.float32)
        e = hist_ref[0, :] / float(_NTOK)
        perp = jnp.exp(-jnp.sum(e * jnp.log(e + 1e-10)))
        perp_ref[...] = jnp.full((1, 1), perp, dtype=jnp.float32)


_HPB = _TOK // _HW


def _quantize_one(z, w):
    zt = jnp.transpose(z, (0, 2, 3, 1)).reshape(_NTOK, _ED)
    zsq = jnp.sum(zt ** 2, axis=1, keepdims=True)
    wsq = jnp.sum(w ** 2, axis=1).reshape(1, _NE)
    out = pl.pallas_call(
        _vq_body,
        grid=(_NBLK,),
        in_specs=[
            pl.BlockSpec((_TOK, _ED), lambda i: (i, 0)),
            pl.BlockSpec((_NE, _ED), lambda i: (0, 0)),
            pl.BlockSpec((_TOK, 1), lambda i: (i, 0)),
            pl.BlockSpec((1, _NE), lambda i: (0, 0)),
        ],
        out_specs=[
            pl.BlockSpec((1, 1, _TOK), lambda i: (i, 0, 0)),
            pl.BlockSpec((1, _ED, _HPB, _HW),
                         lambda i: (i // (_HW // _HPB), 0, i % (_HW // _HPB), 0)),
            pl.BlockSpec((1, 1), lambda i: (0, 0)),
            pl.BlockSpec((1, 1), lambda i: (0, 0)),
        ],
        out_shape=[
            jax.ShapeDtypeStruct((_NBLK, 1, _TOK), jnp.int32),
            jax.ShapeDtypeStruct((_B, _ED, _HW, _HW), jnp.float32),
            jax.ShapeDtypeStruct((1, 1), jnp.float32),
            jax.ShapeDtypeStruct((1, 1), jnp.float32),
        ],
        scratch_shapes=[
            pltpu.VMEM((1, _NE), jnp.float32),
            pltpu.SMEM((1,), jnp.float32),
        ],
    )(zt, w, zsq, wsq)
    idx, zq, loss, perp = out
    return idx.reshape(_NTOK), zq, loss[0, 0], perp[0, 0]


def kernel(z0, z1, W_z, W):
    i0, zq0, l0, p0 = _quantize_one(z0, W_z)
    i1, zq1, l1, p1 = _quantize_one(z1, W)
    return (l0 + l1, zq0, zq1, (p0 + p1) / 2.0, i0, i1)
